# baseline (speedup 1.0000x reference)
import functools

import jax
import jax.numpy as jnp
from jax import lax
from jax.experimental import pallas as pl
from jax.experimental.pallas import tpu as pltpu
from jax.experimental.pallas import tpu_sc as plsc

N = 10000
E = 320000
D = 128
G = 64

NC = 2
NS = 16
NW = NC * NS

NP = 10240
DUMMY = N
C = 64
K = 160
NB = 4
EP = NW * K * C
GP = 72
PB = 40
PK = NP // NW // PB
RPS = NP // NS

_mesh = plsc.VectorSubcoreMesh(core_axis_name="c", subcore_axis_name="s")


HR = NP // D


@functools.partial(
    pl.kernel,
    out_type=jax.ShapeDtypeStruct((NC, HR, D), jnp.float32),
    mesh=_mesh,
    scratch_types=[
        pltpu.VMEM((K, C), jnp.int32),
        pltpu.VMEM((HR, D), jnp.float32),
        pltpu.VMEM((1, HR), jnp.int32),
        pltpu.VMEM_SHARED((HR, D), jnp.float32),
        pltpu.SemaphoreType.DMA,
    ],
    compiler_params=pltpu.CompilerParams(needs_layout_passes=False),
)
def _sc_hist(dst_hbm, zeros_hbm, iota_hbm, out_hbm,
             didx, local_h, idxv, hist_s, sem):
    c = lax.axis_index("c")
    s = lax.axis_index("s")
    wid = s * NC + c
    pltpu.sync_copy(dst_hbm.at[pl.ds(wid * K, K)], didx)
    pltpu.sync_copy(iota_hbm, idxv)
    pltpu.sync_copy(zeros_hbm.at[pl.ds(0, HR)], local_h)

    @pl.when(s == 0)
    def _():
        pltpu.sync_copy(zeros_hbm.at[pl.ds(0, HR)], hist_s)

    ones16 = jnp.full((16,), 1.0, jnp.float32)

    def body(t, carry):
        j = t // (C // 16)
        k = t % (C // 16)
        v = didx[j, pl.ds(k * 16, 16)]
        plsc.addupdate_scatter(local_h, [v >> 7, v & 127], ones16)
        return carry

    lax.fori_loop(0, K * (C // 16), body, 0)
    plsc.subcore_barrier()
    pltpu.sync_copy(local_h, hist_s.at[idxv.at[0]], add=True)
    plsc.subcore_barrier()

    @pl.when(s < HR // 8)
    def _():
        pltpu.sync_copy(hist_s.at[pl.ds(s * 8, 8)],
                        out_hbm.at[c, pl.ds(s * 8, 8)])


@functools.partial(
    pl.kernel,
    out_type=jax.ShapeDtypeStruct((NC, NP, D), jnp.float32),
    mesh=_mesh,
    scratch_types=[
        pltpu.VMEM((K // 4, C), jnp.int32),
        pltpu.VMEM((K // 4, C), jnp.int32),
        [pltpu.VMEM((C, D), jnp.float32)] * NB,
        pltpu.VMEM_SHARED((NP, D), jnp.float32),
        [pltpu.SemaphoreType.DMA] * NB,
        [pltpu.SemaphoreType.DMA] * NB,
    ],
)
def _sc_edge(g_hbm, src_hbm, dst_hbm, zeros_hbm, out_hbm,
             sidx, didx, rows, acc_s, sg, ss):
    c = lax.axis_index("c")
    s = lax.axis_index("s")
    wid = s * NC + c
    KH = K // 4
    pltpu.sync_copy(zeros_hbm.at[pl.ds(s * RPS, RPS)],
                    acc_s.at[pl.ds(s * RPS, RPS)])
    plsc.subcore_barrier()

    for p in range(4):
        pltpu.sync_copy(src_hbm.at[pl.ds(wid * K + p * KH, KH)], sidx)
        pltpu.sync_copy(dst_hbm.at[pl.ds(wid * K + p * KH, KH)], didx)
        for b in range(NB - 1):
            pltpu.async_copy(g_hbm.at[sidx.at[b]], rows[b], sg[b])

        def body(jj, carry):
            m0 = NB * jj
            for b in range(NB):
                m = m0 + b
                pltpu.make_async_copy(g_hbm.at[sidx.at[m]], rows[b],
                                      sg[b]).wait()
                pltpu.async_copy(rows[b], acc_s.at[didx.at[m]], ss[b],
                                 add=True)
                bn = (b + NB - 1) % NB

                @pl.when(m + NB - 1 < KH)
                def _():
                    @pl.when(m > 0)
                    def _():
                        pltpu.make_async_copy(
                            rows[bn], acc_s.at[didx.at[m - 1]], ss[bn]).wait()

                    pltpu.async_copy(g_hbm.at[sidx.at[m + NB - 1]], rows[bn],
                                     sg[bn])
            return carry

        lax.fori_loop(0, KH // NB, body, 0)
        for b in range(NB):
            pltpu.make_async_copy(rows[b], acc_s.at[didx.at[b]], ss[b]).wait()
    plsc.subcore_barrier()
    pltpu.sync_copy(acc_s.at[pl.ds(s * RPS, RPS)],
                    out_hbm.at[c, pl.ds(s * RPS, RPS)])


@functools.partial(
    pl.kernel,
    out_type=[
        jax.ShapeDtypeStruct((NC, GP, D), jnp.float32),
        jax.ShapeDtypeStruct((NC, GP, D), jnp.float32),
    ],
    mesh=_mesh,
    scratch_types=[
        pltpu.VMEM((PK, PB), jnp.int32),
        pltpu.VMEM((PB, D), jnp.float32),
        pltpu.VMEM((PB, D), jnp.float32),
        pltpu.VMEM_SHARED((GP, D), jnp.float32),
        pltpu.VMEM_SHARED((GP, D), jnp.float32),
        pltpu.SemaphoreType.DMA,
    ],
)
def _sc_pool(out_nodes_hbm, batch_hbm, zeros_hbm, ones_hbm,
             sums_hbm, cnt_hbm, bidx, rows, ones_v, sums_s, cnt_s, sem):
    c = lax.axis_index("c")
    s = lax.axis_index("s")
    wid = s * NC + c
    pltpu.sync_copy(batch_hbm.at[pl.ds(wid * PK, PK)], bidx)
    pltpu.sync_copy(ones_hbm.at[pl.ds(0, PB)], ones_v)

    @pl.when(s < GP // 8)
    def _():
        pltpu.sync_copy(zeros_hbm.at[pl.ds(s * 8, 8)], sums_s.at[pl.ds(s * 8, 8)])
        pltpu.sync_copy(zeros_hbm.at[pl.ds(s * 8, 8)], cnt_s.at[pl.ds(s * 8, 8)])

    plsc.subcore_barrier()

    def body(j, carry):
        pltpu.async_copy(
            out_nodes_hbm.at[pl.ds(wid * (PK * PB) + j * PB, PB)], rows, sem
        ).wait()
        pltpu.sync_copy(rows, sums_s.at[bidx.at[j]], add=True)
        pltpu.sync_copy(ones_v, cnt_s.at[bidx.at[j]], add=True)
        return carry

    lax.fori_loop(0, PK, body, 0)
    plsc.subcore_barrier()

    @pl.when(s < GP // 8)
    def _():
        pltpu.sync_copy(sums_s.at[pl.ds(s * 8, 8)], sums_hbm.at[c, pl.ds(s * 8, 8)])
        pltpu.sync_copy(cnt_s.at[pl.ds(s * 8, 8)], cnt_hbm.at[c, pl.ds(s * 8, 8)])


BM = 1024
GRID = NP // BM

_acc_spec = pl.BlockSpec((NC, BM, D), lambda j: (0, j, 0))
_hist_spec = pl.BlockSpec((NC, BM // D, D), lambda j: (0, j, 0))
_row_spec = pl.BlockSpec((BM, D), lambda j: (j, 0))
_w_spec = pl.BlockSpec((D, D), lambda j: (0, 0))
_vec_spec = pl.BlockSpec((1, D), lambda j: (0, 0))


def _dinv_of(hist_ref):
    cnt2 = hist_ref[0] + hist_ref[1]
    rep = jnp.concatenate(
        [jnp.broadcast_to(cnt2[q:q + 1, :], (D, D)) for q in range(BM // D)],
        axis=0)
    row = lax.broadcasted_iota(jnp.int32, (BM, D), 0)
    lane = lax.broadcasted_iota(jnp.int32, (BM, D), 1)
    sel = jnp.where((row % D) == lane, rep, 0.0)
    cnt = jnp.sum(sel, axis=1, keepdims=True)
    return lax.rsqrt(cnt + 1.0)


def _tc_stage1_body(hist_ref, x_ref, w1_ref, o_ref):
    dinv = _dinv_of(hist_ref)
    h = jnp.dot(x_ref[...], w1_ref[...], preferred_element_type=jnp.float32)
    o_ref[...] = h * dinv


def _tc_stage2_body(hist_ref, a1_ref, g1_ref, b1_ref, gamma_ref, beta_ref,
                    rm_ref, rv_ref, w2_ref, o_ref):
    dinv = _dinv_of(hist_ref)
    out1 = dinv * (a1_ref[0] + a1_ref[1] + g1_ref[...]) + b1_ref[...]
    scale = gamma_ref[...] * lax.rsqrt(rv_ref[...] + 1e-5)
    bn = (out1 - rm_ref[...]) * scale + beta_ref[...]
    e = jnp.where(bn > 0, bn, jnp.exp(bn) - 1.0)
    h = jnp.dot(e, w2_ref[...], preferred_element_type=jnp.float32)
    o_ref[...] = h * dinv


def _tc_stage3_body(hist_ref, a2_ref, g2_ref, b2_ref, o_ref):
    dinv = _dinv_of(hist_ref)
    o_ref[...] = dinv * (a2_ref[0] + a2_ref[1] + g2_ref[...]) + b2_ref[...]


def _tc_final_body(sums_ref, cnt_ref, o_ref):
    ssum = sums_ref[0] + sums_ref[1]
    csum = cnt_ref[0] + cnt_ref[1]
    rep = ssum / jnp.maximum(csum, 1.0)
    o_ref[...] = rep[:G]


_tc_stage1 = pl.pallas_call(
    _tc_stage1_body,
    grid=(GRID,),
    in_specs=[_hist_spec, _row_spec, _w_spec],
    out_specs=_row_spec,
    out_shape=jax.ShapeDtypeStruct((NP, D), jnp.float32),
)

_tc_stage2 = pl.pallas_call(
    _tc_stage2_body,
    grid=(GRID,),
    in_specs=[_hist_spec, _acc_spec, _row_spec, _vec_spec, _vec_spec,
              _vec_spec, _vec_spec, _vec_spec, _w_spec],
    out_specs=_row_spec,
    out_shape=jax.ShapeDtypeStruct((NP, D), jnp.float32),
)

_tc_stage3 = pl.pallas_call(
    _tc_stage3_body,
    grid=(GRID,),
    in_specs=[_hist_spec, _acc_spec, _row_spec, _vec_spec],
    out_specs=_row_spec,
    out_shape=jax.ShapeDtypeStruct((NP, D), jnp.float32),
)

_tc_final = pl.pallas_call(
    _tc_final_body,
    in_specs=[pl.BlockSpec((NC, GP, D), lambda: (0, 0, 0)),
              pl.BlockSpec((NC, GP, D), lambda: (0, 0, 0))],
    out_specs=pl.BlockSpec((G, D), lambda: (0, 0)),
    out_shape=jax.ShapeDtypeStruct((G, D), jnp.float32),
)


def kernel(x, edge_index, batch, W1, b1, gamma, beta, rm, rv, W2, b2):
    src = edge_index[0]
    dst = edge_index[1]
    pad_e = N + jnp.arange(EP - E, dtype=jnp.int32) % (NP - N)
    src_p = jnp.concatenate([src, pad_e]).reshape(NW * K, C)
    dst_p = jnp.concatenate([dst, pad_e]).reshape(NW * K, C)
    batch_p = jnp.concatenate(
        [batch, jnp.full((NP - N,), G, dtype=jnp.int32)]
    ).reshape(NP // PB, PB)
    x_p = jnp.pad(x, ((0, NP - N), (0, 0)))

    zeros = jnp.zeros((NP, D), jnp.float32)
    ones = jnp.ones((C, D), jnp.float32)
    iota_h = jnp.arange(HR, dtype=jnp.int32).reshape(1, HR)

    hist = _sc_hist(dst_p, zeros, iota_h)

    g1 = _tc_stage1(hist, x_p, W1)
    a1 = _sc_edge(g1, src_p, dst_p, zeros)
    g2 = _tc_stage2(hist, a1, g1, b1.reshape(1, D), gamma.reshape(1, D),
                    beta.reshape(1, D), rm.reshape(1, D), rv.reshape(1, D), W2)
    a2 = _sc_edge(g2, src_p, dst_p, zeros)
    out_p = _tc_stage3(hist, a2, g2, b2.reshape(1, D))

    sums, cnt = _sc_pool(out_p, batch_p, zeros, ones)
    graph_rep = _tc_final(sums, cnt)
    return out_p[:N], graph_rep


# speedup vs baseline: 1.0810x; 1.0810x over previous
import functools

import jax
import jax.numpy as jnp
from jax import lax
from jax.experimental import pallas as pl
from jax.experimental.pallas import tpu as pltpu
from jax.experimental.pallas import tpu_sc as plsc

N = 10000
E = 320000
D = 128
G = 64

NC = 2
NS = 16
NW = NC * NS

NP = 10240
DUMMY = N
C = 128
K = 80
EP = NW * K * C
RPS = NP // NS

_mesh = plsc.VectorSubcoreMesh(core_axis_name="c", subcore_axis_name="s")


HR = NP // D


@functools.partial(
    pl.kernel,
    out_type=jax.ShapeDtypeStruct((NC, HR, D), jnp.float32),
    mesh=_mesh,
    scratch_types=[
        pltpu.VMEM((K, C), jnp.int32),
        pltpu.VMEM((HR, D), jnp.float32),
        pltpu.VMEM((1, HR), jnp.int32),
        pltpu.VMEM_SHARED((HR, D), jnp.float32),
        pltpu.SemaphoreType.DMA,
    ],
    compiler_params=pltpu.CompilerParams(needs_layout_passes=False),
)
def _sc_hist(dst_hbm, zeros_hbm, iota_hbm, out_hbm,
             didx, local_h, idxv, hist_s, sem):
    c = lax.axis_index("c")
    s = lax.axis_index("s")
    wid = s * NC + c
    pltpu.sync_copy(dst_hbm.at[pl.ds(wid * K, K)], didx)
    pltpu.sync_copy(iota_hbm, idxv)
    pltpu.sync_copy(zeros_hbm.at[pl.ds(0, HR)], local_h)

    @pl.when(s == 0)
    def _():
        pltpu.sync_copy(zeros_hbm.at[pl.ds(0, HR)], hist_s)

    ones16 = jnp.full((16,), 1.0, jnp.float32)

    def body(t, carry):
        j = t // (C // 16)
        k = t % (C // 16)
        v = didx[j, pl.ds(k * 16, 16)]
        plsc.addupdate_scatter(local_h, [v >> 7, v & 127], ones16)
        return carry

    lax.fori_loop(0, K * (C // 16), body, 0)
    plsc.subcore_barrier()
    pltpu.sync_copy(local_h, hist_s.at[idxv.at[0]], add=True)
    plsc.subcore_barrier()

    @pl.when(s < HR // 8)
    def _():
        pltpu.sync_copy(hist_s.at[pl.ds(s * 8, 8)],
                        out_hbm.at[c, pl.ds(s * 8, 8)])


@functools.partial(
    pl.kernel,
    out_type=jax.ShapeDtypeStruct((NC, NP, D), jnp.float32),
    mesh=_mesh,
    scratch_types=[
        pltpu.VMEM((K // 2, C), jnp.int32),
        pltpu.VMEM((K // 2, C), jnp.int32),
        pltpu.VMEM((C, D), jnp.float32),
        pltpu.VMEM((C, D), jnp.float32),
        pltpu.VMEM_SHARED((NP, D), jnp.float32),
        pltpu.SemaphoreType.DMA,
        pltpu.SemaphoreType.DMA,
    ],
)
def _sc_edge(g_hbm, src_hbm, dst_hbm, zeros_hbm, out_hbm,
             sidx, didx, rows, rows2, acc_s, sem, sem2):
    c = lax.axis_index("c")
    s = lax.axis_index("s")
    wid = s * NC + c
    KH = K // 2

    @pl.when(c == 0)
    def _():
        pltpu.sync_copy(g_hbm.at[pl.ds(s * RPS, RPS)],
                        acc_s.at[pl.ds(s * RPS, RPS)])

    @pl.when(c != 0)
    def _():
        pltpu.sync_copy(zeros_hbm.at[pl.ds(s * RPS, RPS)],
                        acc_s.at[pl.ds(s * RPS, RPS)])

    plsc.subcore_barrier()

    for p in range(2):
        pltpu.sync_copy(src_hbm.at[pl.ds(wid * K + p * KH, KH)], sidx)
        pltpu.sync_copy(dst_hbm.at[pl.ds(wid * K + p * KH, KH)], didx)
        pltpu.async_copy(g_hbm.at[sidx.at[0]], rows, sem)

        def body(jj, carry):
            j = 2 * jj
            pltpu.async_copy(g_hbm.at[sidx.at[j + 1]], rows2, sem2)
            pltpu.make_async_copy(g_hbm.at[sidx.at[j]], rows, sem).wait()
            pltpu.sync_copy(rows, acc_s.at[didx.at[j]], add=True)

            @pl.when(jj < KH // 2 - 1)
            def _():
                pltpu.async_copy(g_hbm.at[sidx.at[j + 2]], rows, sem)

            pltpu.make_async_copy(g_hbm.at[sidx.at[j + 1]], rows2, sem2).wait()
            pltpu.sync_copy(rows2, acc_s.at[didx.at[j + 1]], add=True)
            return carry

        lax.fori_loop(0, KH // 2, body, 0)
    plsc.subcore_barrier()
    pltpu.sync_copy(acc_s.at[pl.ds(s * RPS, RPS)],
                    out_hbm.at[c, pl.ds(s * RPS, RPS)])


BM = 1024
GRID = NP // BM

_acc_spec = pl.BlockSpec((NC, BM, D), lambda j: (0, j, 0))
_hist_spec = pl.BlockSpec((NC, BM // D, D), lambda j: (0, j, 0))
_row_spec = pl.BlockSpec((BM, D), lambda j: (j, 0))
_w_spec = pl.BlockSpec((D, D), lambda j: (0, 0))
_vec_spec = pl.BlockSpec((1, D), lambda j: (0, 0))


def _dinv_of(hist_ref):
    cnt = _expand_packed(hist_ref[0] + hist_ref[1], 0.0)
    return lax.rsqrt(cnt + 1.0)


def _tc_stage1_body(hist_ref, x_ref, w1_ref, o_ref):
    dinv = _dinv_of(hist_ref)
    h = jnp.dot(x_ref[...], w1_ref[...], preferred_element_type=jnp.float32)
    o_ref[...] = h * dinv


def _tc_stage2_body(hist_ref, a1_ref, b1_ref, gamma_ref, beta_ref,
                    rm_ref, rv_ref, w2_ref, o_ref):
    dinv = _dinv_of(hist_ref)
    out1 = dinv * (a1_ref[0] + a1_ref[1]) + b1_ref[...]
    scale = gamma_ref[...] * lax.rsqrt(rv_ref[...] + 1e-5)
    bn = (out1 - rm_ref[...]) * scale + beta_ref[...]
    e = jnp.where(bn > 0, bn, jnp.exp(bn) - 1.0)
    h = jnp.dot(e, w2_ref[...], preferred_element_type=jnp.float32)
    o_ref[...] = h * dinv


def _expand_packed(pk, zero):
    rep = jnp.concatenate(
        [jnp.broadcast_to(pk[q:q + 1, :], (D, D)) for q in range(BM // D)],
        axis=0)
    row = lax.broadcasted_iota(jnp.int32, (BM, D), 0)
    lane = lax.broadcasted_iota(jnp.int32, (BM, D), 1)
    sel = jnp.where((row % D) == lane, rep, zero)
    return jnp.sum(sel, axis=1, keepdims=True)


def _tc_stage3_body(hist_ref, a2_ref, b2_ref, batch_ref,
                    o_ref, grep_ref, sums_acc, cnt_acc):
    j = pl.program_id(0)
    dinv = _dinv_of(hist_ref)
    out = dinv * (a2_ref[0] + a2_ref[1]) + b2_ref[...]
    o_ref[...] = out

    bcol = _expand_packed(batch_ref[...], 0)
    gid = lax.broadcasted_iota(jnp.int32, (BM, G), 1)
    p = jnp.where(gid == bcol, 1.0, 0.0)
    dn = (((0,), (0,)), ((), ()))
    psum = lax.dot_general(p, out, dn,
                           preferred_element_type=jnp.float32)
    pcnt = lax.dot_general(p, jnp.ones((BM, D), jnp.float32), dn,
                           preferred_element_type=jnp.float32)

    @pl.when(j == 0)
    def _():
        sums_acc[...] = psum
        cnt_acc[...] = pcnt

    @pl.when(j > 0)
    def _():
        sums_acc[...] += psum
        cnt_acc[...] += pcnt

    grep_ref[...] = sums_acc[...] / jnp.maximum(cnt_acc[...], 1.0)


_tc_stage1 = pl.pallas_call(
    _tc_stage1_body,
    grid=(GRID,),
    in_specs=[_hist_spec, _row_spec, _w_spec],
    out_specs=_row_spec,
    out_shape=jax.ShapeDtypeStruct((NP, D), jnp.float32),
)

_tc_stage2 = pl.pallas_call(
    _tc_stage2_body,
    grid=(GRID,),
    in_specs=[_hist_spec, _acc_spec, _vec_spec, _vec_spec,
              _vec_spec, _vec_spec, _vec_spec, _w_spec],
    out_specs=_row_spec,
    out_shape=jax.ShapeDtypeStruct((NP, D), jnp.float32),
)

_tc_stage3 = pl.pallas_call(
    _tc_stage3_body,
    grid=(GRID,),
    in_specs=[_hist_spec, _acc_spec, _vec_spec,
              pl.BlockSpec((BM // D, D), lambda j: (j, 0))],
    out_specs=[_row_spec, pl.BlockSpec((G, D), lambda j: (0, 0))],
    out_shape=[jax.ShapeDtypeStruct((NP, D), jnp.float32),
               jax.ShapeDtypeStruct((G, D), jnp.float32)],
    scratch_shapes=[pltpu.VMEM((G, D), jnp.float32),
                    pltpu.VMEM((G, D), jnp.float32)],
)


def kernel(x, edge_index, batch, W1, b1, gamma, beta, rm, rv, W2, b2):
    src = edge_index[0]
    dst = edge_index[1]
    pad_e = N + jnp.arange(EP - E, dtype=jnp.int32) % (NP - N)
    src_p = jnp.concatenate([src, pad_e]).reshape(NW * K, C)
    dst_p = jnp.concatenate([dst, pad_e]).reshape(NW * K, C)
    batch_pk = jnp.concatenate(
        [batch, jnp.full((NP - N,), G, dtype=jnp.int32)]
    ).reshape(HR, D)
    x_p = jnp.pad(x, ((0, NP - N), (0, 0)))

    zeros = jnp.zeros((NP, D), jnp.float32)
    iota_h = jnp.arange(HR, dtype=jnp.int32).reshape(1, HR)

    hist = _sc_hist(dst_p, zeros, iota_h)

    g1 = _tc_stage1(hist, x_p, W1)
    a1 = _sc_edge(g1, src_p, dst_p, zeros)
    g2 = _tc_stage2(hist, a1, b1.reshape(1, D), gamma.reshape(1, D),
                    beta.reshape(1, D), rm.reshape(1, D), rv.reshape(1, D), W2)
    a2 = _sc_edge(g2, src_p, dst_p, zeros)
    out_p, graph_rep = _tc_stage3(hist, a2, b2.reshape(1, D), batch_pk)
    return out_p[:N], graph_rep


# speedup vs baseline: 1.0817x; 1.0006x over previous
import functools

import jax
import jax.numpy as jnp
from jax import lax
from jax.experimental import pallas as pl
from jax.experimental.pallas import tpu as pltpu
from jax.experimental.pallas import tpu_sc as plsc

N = 10000
E = 320000
D = 128
G = 64

NC = 2
NS = 16
NW = NC * NS

NP = 10240
DUMMY = N
C = 128
K = 80
EP = NW * K * C
RPS = NP // NS

_mesh = plsc.VectorSubcoreMesh(core_axis_name="c", subcore_axis_name="s")


HR = NP // D


@functools.partial(
    pl.kernel,
    out_type=jax.ShapeDtypeStruct((NC, HR, D), jnp.float32),
    mesh=_mesh,
    scratch_types=[
        pltpu.VMEM((K, C), jnp.int32),
        pltpu.VMEM((HR, D), jnp.float32),
        pltpu.VMEM((1, HR), jnp.int32),
        pltpu.VMEM_SHARED((HR, D), jnp.float32),
        pltpu.SemaphoreType.DMA,
    ],
    compiler_params=pltpu.CompilerParams(needs_layout_passes=False),
)
def _sc_hist(dst_hbm, zeros_hbm, iota_hbm, out_hbm,
             didx, local_h, idxv, hist_s, sem):
    c = lax.axis_index("c")
    s = lax.axis_index("s")
    wid = s * NC + c
    pltpu.sync_copy(dst_hbm.at[pl.ds(wid * K, K)], didx)
    pltpu.sync_copy(iota_hbm, idxv)
    pltpu.sync_copy(zeros_hbm.at[pl.ds(0, HR)], local_h)

    @pl.when(s == 0)
    def _():
        pltpu.sync_copy(zeros_hbm.at[pl.ds(0, HR)], hist_s)

    ones16 = jnp.full((16,), 1.0, jnp.float32)

    def body(t, carry):
        j = t // (C // 16)
        k = t % (C // 16)
        v = didx[j, pl.ds(k * 16, 16)]
        plsc.addupdate_scatter(local_h, [v >> 7, v & 127], ones16)
        return carry

    lax.fori_loop(0, K * (C // 16), body, 0)
    plsc.subcore_barrier()
    pltpu.sync_copy(local_h, hist_s.at[idxv.at[0]], add=True)
    plsc.subcore_barrier()

    @pl.when(s < HR // 8)
    def _():
        pltpu.sync_copy(hist_s.at[pl.ds(s * 8, 8)],
                        out_hbm.at[c, pl.ds(s * 8, 8)])


@functools.partial(
    pl.kernel,
    out_type=jax.ShapeDtypeStruct((NC, NP, D), jnp.float32),
    mesh=_mesh,
    scratch_types=[
        pltpu.VMEM((K // 2, C), jnp.int32),
        pltpu.VMEM((K // 2, C), jnp.int32),
        pltpu.VMEM((C, D), jnp.float32),
        pltpu.VMEM((C, D), jnp.float32),
        pltpu.VMEM_SHARED((NP, D), jnp.float32),
        pltpu.SemaphoreType.DMA,
        pltpu.SemaphoreType.DMA,
        pltpu.SemaphoreType.DMA,
    ],
)
def _sc_edge(g_hbm, src_hbm, dst_hbm, zeros_hbm, out_hbm,
             sidx, didx, rows, rows2, acc_s, sem, sem2, ss2):
    c = lax.axis_index("c")
    s = lax.axis_index("s")
    wid = s * NC + c
    KH = K // 2

    @pl.when(c == 0)
    def _():
        pltpu.sync_copy(g_hbm.at[pl.ds(s * RPS, RPS)],
                        acc_s.at[pl.ds(s * RPS, RPS)])

    @pl.when(c != 0)
    def _():
        pltpu.sync_copy(zeros_hbm.at[pl.ds(s * RPS, RPS)],
                        acc_s.at[pl.ds(s * RPS, RPS)])

    plsc.subcore_barrier()

    for p in range(2):
        pltpu.sync_copy(src_hbm.at[pl.ds(wid * K + p * KH, KH)], sidx)
        pltpu.sync_copy(dst_hbm.at[pl.ds(wid * K + p * KH, KH)], didx)
        pltpu.async_copy(g_hbm.at[sidx.at[0]], rows, sem)

        def body(jj, carry):
            j = 2 * jj

            @pl.when(jj > 0)
            def _():
                pltpu.make_async_copy(rows2, acc_s.at[didx.at[j - 1]],
                                      ss2).wait()

            pltpu.async_copy(g_hbm.at[sidx.at[j + 1]], rows2, sem2)
            pltpu.make_async_copy(g_hbm.at[sidx.at[j]], rows, sem).wait()
            pltpu.sync_copy(rows, acc_s.at[didx.at[j]], add=True)

            @pl.when(jj < KH // 2 - 1)
            def _():
                pltpu.async_copy(g_hbm.at[sidx.at[j + 2]], rows, sem)

            pltpu.make_async_copy(g_hbm.at[sidx.at[j + 1]], rows2, sem2).wait()
            pltpu.async_copy(rows2, acc_s.at[didx.at[j + 1]], ss2, add=True)
            return carry

        lax.fori_loop(0, KH // 2, body, 0)
        pltpu.make_async_copy(rows2, acc_s.at[didx.at[KH - 1]], ss2).wait()
    plsc.subcore_barrier()
    pltpu.sync_copy(acc_s.at[pl.ds(s * RPS, RPS)],
                    out_hbm.at[c, pl.ds(s * RPS, RPS)])


BM = 1024
GRID = NP // BM

_acc_spec = pl.BlockSpec((NC, BM, D), lambda j: (0, j, 0))
_hist_spec = pl.BlockSpec((NC, BM // D, D), lambda j: (0, j, 0))
_row_spec = pl.BlockSpec((BM, D), lambda j: (j, 0))
_w_spec = pl.BlockSpec((D, D), lambda j: (0, 0))
_vec_spec = pl.BlockSpec((1, D), lambda j: (0, 0))


def _dinv_of(hist_ref):
    cnt = _expand_packed(hist_ref[0] + hist_ref[1], 0.0)
    return lax.rsqrt(cnt + 1.0)


def _tc_stage1_body(hist_ref, x_ref, w1_ref, o_ref):
    dinv = _dinv_of(hist_ref)
    h = jnp.dot(x_ref[...], w1_ref[...], preferred_element_type=jnp.float32)
    o_ref[...] = h * dinv


def _tc_stage2_body(hist_ref, a1_ref, b1_ref, gamma_ref, beta_ref,
                    rm_ref, rv_ref, w2_ref, o_ref):
    dinv = _dinv_of(hist_ref)
    out1 = dinv * (a1_ref[0] + a1_ref[1]) + b1_ref[...]
    scale = gamma_ref[...] * lax.rsqrt(rv_ref[...] + 1e-5)
    bn = (out1 - rm_ref[...]) * scale + beta_ref[...]
    e = jnp.where(bn > 0, bn, jnp.exp(bn) - 1.0)
    h = jnp.dot(e, w2_ref[...], preferred_element_type=jnp.float32)
    o_ref[...] = h * dinv


def _expand_packed(pk, zero):
    rep = jnp.concatenate(
        [jnp.broadcast_to(pk[q:q + 1, :], (D, D)) for q in range(BM // D)],
        axis=0)
    row = lax.broadcasted_iota(jnp.int32, (BM, D), 0)
    lane = lax.broadcasted_iota(jnp.int32, (BM, D), 1)
    sel = jnp.where((row % D) == lane, rep, zero)
    return jnp.sum(sel, axis=1, keepdims=True)


def _tc_stage3_body(hist_ref, a2_ref, b2_ref, batch_ref,
                    o_ref, grep_ref, sums_acc, cnt_acc):
    j = pl.program_id(0)
    dinv = _dinv_of(hist_ref)
    out = dinv * (a2_ref[0] + a2_ref[1]) + b2_ref[...]
    o_ref[...] = out

    bcol = _expand_packed(batch_ref[...], 0)
    gid = lax.broadcasted_iota(jnp.int32, (BM, G), 1)
    p = jnp.where(gid == bcol, 1.0, 0.0)
    dn = (((0,), (0,)), ((), ()))
    psum = lax.dot_general(p, out, dn,
                           preferred_element_type=jnp.float32)
    pcnt = lax.dot_general(p, jnp.ones((BM, D), jnp.float32), dn,
                           preferred_element_type=jnp.float32)

    @pl.when(j == 0)
    def _():
        sums_acc[...] = psum
        cnt_acc[...] = pcnt

    @pl.when(j > 0)
    def _():
        sums_acc[...] += psum
        cnt_acc[...] += pcnt

    grep_ref[...] = sums_acc[...] / jnp.maximum(cnt_acc[...], 1.0)


_tc_stage1 = pl.pallas_call(
    _tc_stage1_body,
    grid=(GRID,),
    in_specs=[_hist_spec, _row_spec, _w_spec],
    out_specs=_row_spec,
    out_shape=jax.ShapeDtypeStruct((NP, D), jnp.float32),
)

_tc_stage2 = pl.pallas_call(
    _tc_stage2_body,
    grid=(GRID,),
    in_specs=[_hist_spec, _acc_spec, _vec_spec, _vec_spec,
              _vec_spec, _vec_spec, _vec_spec, _w_spec],
    out_specs=_row_spec,
    out_shape=jax.ShapeDtypeStruct((NP, D), jnp.float32),
)

_tc_stage3 = pl.pallas_call(
    _tc_stage3_body,
    grid=(GRID,),
    in_specs=[_hist_spec, _acc_spec, _vec_spec,
              pl.BlockSpec((BM // D, D), lambda j: (j, 0))],
    out_specs=[_row_spec, pl.BlockSpec((G, D), lambda j: (0, 0))],
    out_shape=[jax.ShapeDtypeStruct((NP, D), jnp.float32),
               jax.ShapeDtypeStruct((G, D), jnp.float32)],
    scratch_shapes=[pltpu.VMEM((G, D), jnp.float32),
                    pltpu.VMEM((G, D), jnp.float32)],
)


def kernel(x, edge_index, batch, W1, b1, gamma, beta, rm, rv, W2, b2):
    src = edge_index[0]
    dst = edge_index[1]
    pad_e = N + jnp.arange(EP - E, dtype=jnp.int32) % (NP - N)
    src_p = jnp.concatenate([src, pad_e]).reshape(NW * K, C)
    dst_p = jnp.concatenate([dst, pad_e]).reshape(NW * K, C)
    batch_pk = jnp.concatenate(
        [batch, jnp.full((NP - N,), G, dtype=jnp.int32)]
    ).reshape(HR, D)
    x_p = jnp.pad(x, ((0, NP - N), (0, 0)))

    zeros = jnp.zeros((NP, D), jnp.float32)
    iota_h = jnp.arange(HR, dtype=jnp.int32).reshape(1, HR)

    hist = _sc_hist(dst_p, zeros, iota_h)

    g1 = _tc_stage1(hist, x_p, W1)
    a1 = _sc_edge(g1, src_p, dst_p, zeros)
    g2 = _tc_stage2(hist, a1, b1.reshape(1, D), gamma.reshape(1, D),
                    beta.reshape(1, D), rm.reshape(1, D), rv.reshape(1, D), W2)
    a2 = _sc_edge(g2, src_p, dst_p, zeros)
    out_p, graph_rep = _tc_stage3(hist, a2, b2.reshape(1, D), batch_pk)
    return out_p[:N], graph_rep
